# baseline (speedup 1.0000x reference)
import functools

import jax
import jax.numpy as jnp
from jax import lax
from jax.experimental import pallas as pl
from jax.experimental.pallas import tpu as pltpu
from jax.experimental.pallas import tpu_sc as plsc

N = 50000
H = 96
E = 800000
NC, NS = 2, 16
NW = NC * NS
EPW = E // NW
BE = 200
IDXB = 5000
NBLK = EPW // IDXB
SUBB = IDXB // BE
CW = 32
NCH = H // CW
RPT = N // NS
ZR = 125
DPT = 3128
DLAST = N - (NS - 1) * DPT
CNT_PAD = 50048
CPT = CNT_PAD // NS
ZC_LEN = 800
ONE_LEN = 208

IDXB_C = 5000
ONE_LEN_C = 5008

NB = 2000
GRID = N // NB
BN_EPS = 1e-5

_mesh = plsc.VectorSubcoreMesh(core_axis_name="c", subcore_axis_name="s",
                               num_cores=NC, num_subcores=NS)
_sc_params = pltpu.CompilerParams(use_tc_tiling_on_sc=False)
_tc_params = pltpu.CompilerParams(vmem_limit_bytes=100 * 1024 * 1024)


def _zero_vmem_1d(ref, n16):
    z = jnp.zeros((16,), jnp.float32)
    @pl.loop(0, n16)
    def _(i):
        ref[pl.ds(i * 16, 16)] = z


def _zero_cnt_acc(cnt_acc, zcnt, s):
    for off in range(0, CPT, ZC_LEN):
        sz = min(ZC_LEN, CPT - off)
        pltpu.sync_copy(zcnt.at[pl.ds(0, sz)],
                        cnt_acc.at[pl.ds(s * CPT + off, sz)])


def _dump_cnt(cnt_acc, cnt_out, c, s):
    pltpu.sync_copy(cnt_acc.at[pl.ds(s * CPT, CPT)],
                    cnt_out.at[pl.ds(c * CNT_PAD + s * CPT, CPT)])


def _one_rel(with_cnt, xs, esrc, edst, aggs, cnt_out, c, s, wid,
             acc, cnt_acc, zbuf, zcnt, ones, bsrc, bdst, rbufs, sems):
    for k in range(NCH):
        for zz in range(RPT // ZR):
            pltpu.sync_copy(zbuf, acc.at[pl.ds(s * RPT + zz * ZR, ZR), :])
        if with_cnt and k == 0:
            _zero_cnt_acc(cnt_acc, zcnt, s)
        plsc.subcore_barrier()

        cnt_en = with_cnt and k == 0

        @pl.loop(0, NBLK)
        def _(blk):
            base = wid * EPW + blk * IDXB
            pltpu.sync_copy(esrc.at[pl.ds(base, IDXB)], bsrc)
            pltpu.sync_copy(edst.at[pl.ds(base, IDXB)], bdst)

            def scat(t):
                pltpu.sync_copy(rbufs[t % 2],
                                acc.at[bdst.at[pl.ds(t * BE, BE)]],
                                add=True)
                if cnt_en:
                    pltpu.sync_copy(ones.at[pl.ds(0, BE)],
                                    cnt_acc.at[bdst.at[pl.ds(t * BE, BE)]],
                                    add=True)

            descs = [None, None]
            descs[0] = pltpu.async_copy(xs[k].at[bsrc.at[pl.ds(0, BE)]],
                                        rbufs[0], sems[0])
            for t in range(1, SUBB):
                b = t % 2
                descs[b] = pltpu.async_copy(
                    xs[k].at[bsrc.at[pl.ds(t * BE, BE)]], rbufs[b], sems[b])
                descs[1 - b].wait()
                scat(t - 1)
            descs[(SUBB - 1) % 2].wait()
            scat(SUBB - 1)

        plsc.subcore_barrier()
        @pl.when(s < NS - 1)
        def _():
            pltpu.sync_copy(acc.at[pl.ds(s * DPT, DPT), :],
                            aggs[k].at[c, pl.ds(s * DPT, DPT), :])
        @pl.when(s == NS - 1)
        def _():
            pltpu.sync_copy(
                acc.at[pl.ds((NS - 1) * DPT, DLAST), :],
                aggs[k].at[c, pl.ds((NS - 1) * DPT, DLAST), :])
        if with_cnt and k == 0:
            _dump_cnt(cnt_acc, cnt_out, c, s)
        plsc.subcore_barrier()


def _cnt_rel(edst, cnt_out, c, s, wid, cnt_acc, zcnt, ones, bdst):
    _zero_cnt_acc(cnt_acc, zcnt, s)
    plsc.subcore_barrier()

    @pl.loop(0, NBLK)
    def _(blk):
        base = wid * EPW + blk * IDXB
        pltpu.sync_copy(edst.at[pl.ds(base, IDXB)], bdst)
        for t in range(SUBB):
            pltpu.sync_copy(ones.at[pl.ds(0, BE)],
                            cnt_acc.at[bdst.at[pl.ds(t * BE, BE)]],
                            add=True)

    plsc.subcore_barrier()
    _dump_cnt(cnt_acc, cnt_out, c, s)
    plsc.subcore_barrier()


def _make_seg_multi(cnt_flags, n_extra_cnt=0):
    nrel = len(cnt_flags)
    any_cnt = any(cnt_flags) or n_extra_cnt > 0
    out_types = []
    for f in cnt_flags:
        out_types += [jax.ShapeDtypeStruct((NC, N, CW), jnp.float32)] * NCH
        if f:
            out_types.append(
                jax.ShapeDtypeStruct((NC * CNT_PAD,), jnp.float32))
    out_types += [jax.ShapeDtypeStruct((NC * CNT_PAD,), jnp.float32)
                  ] * n_extra_cnt

    def body(*refs):
        pos = 0
        rels = []
        for f in cnt_flags:
            xs = refs[pos:pos + NCH]
            esrc, edst = refs[pos + NCH], refs[pos + NCH + 1]
            pos += NCH + 2
            rels.append((f, xs, esrc, edst))
        extra_edst = refs[pos:pos + n_extra_cnt]
        pos += n_extra_cnt
        outs = []
        for f in cnt_flags:
            aggs = refs[pos:pos + NCH]
            pos += NCH
            cnt_o = refs[pos] if f else None
            pos += 1 if f else 0
            outs.append((aggs, cnt_o))
        extra_cnt_out = refs[pos:pos + n_extra_cnt]
        pos += n_extra_cnt
        (acc, cnt_acc, zbuf, zcnt, ones, bsrc, bdst, rows_a, rows_b,
         sem_a, sem_b) = refs[pos:]
        c = lax.axis_index("c")
        s = lax.axis_index("s")
        wid = s * NC + c

        z = jnp.zeros((16,), jnp.float32)
        @pl.loop(0, ZR)
        def _(i):
            for j in range(CW // 16):
                zbuf[i, pl.ds(j * 16, 16)] = z
        if any_cnt:
            _zero_vmem_1d(zcnt, ZC_LEN // 16)
            o = jnp.ones((16,), jnp.float32)
            @pl.loop(0, ONE_LEN // 16)
            def _(i):
                ones[pl.ds(i * 16, 16)] = o

        for (f, xs, esrc, edst), (aggs, cnt_o) in zip(rels, outs):
            _one_rel(f, xs, esrc, edst, aggs, cnt_o, c, s, wid,
                     acc, cnt_acc, zbuf, zcnt, ones, bsrc, bdst,
                     (rows_a, rows_b), (sem_a, sem_b))
        for edst, cnt_o in zip(extra_edst, extra_cnt_out):
            _cnt_rel(edst, cnt_o, c, s, wid, cnt_acc, zcnt, ones, bdst)

    return pl.kernel(
        body,
        out_type=tuple(out_types),
        mesh=_mesh,
        compiler_params=_sc_params,
        scratch_types=[
            pltpu.VMEM_SHARED((N, CW), jnp.float32),
            pltpu.VMEM_SHARED((CNT_PAD,) if any_cnt else (8,), jnp.float32),
            pltpu.VMEM((ZR, CW), jnp.float32),
            pltpu.VMEM((ZC_LEN if any_cnt else 16,), jnp.float32),
            pltpu.VMEM((ONE_LEN if any_cnt else 16,), jnp.float32),
            pltpu.VMEM((IDXB,), jnp.int32),
            pltpu.VMEM((IDXB,), jnp.int32),
            pltpu.VMEM((BE, CW), jnp.float32),
            pltpu.VMEM((BE, CW), jnp.float32),
            pltpu.SemaphoreType.DMA,
            pltpu.SemaphoreType.DMA,
        ],
    )


_seg_12 = _make_seg_multi((True, True))
_seg_3 = _make_seg_multi((True,))
_seg_45 = _make_seg_multi((False, False))
_seg_67 = _make_seg_multi((True, True))
_seg_8 = _make_seg_multi((True,))
_seg_9 = _make_seg_multi((False,))


def _cnt_body(edst, cnt_out, cnt_acc, zcnt, ones, idx_d):
    c = lax.axis_index("c")
    s = lax.axis_index("s")
    wid = s * NC + c
    _zero_vmem_1d(zcnt, ZC_LEN // 16)
    o = jnp.ones((16,), jnp.float32)
    @pl.loop(0, ONE_LEN_C // 16)
    def _(i):
        ones[pl.ds(i * 16, 16)] = o
    _zero_cnt_acc(cnt_acc, zcnt, s)
    plsc.subcore_barrier()

    @pl.loop(0, EPW // IDXB_C)
    def _(i):
        base = wid * EPW + i * IDXB_C
        pltpu.sync_copy(edst.at[pl.ds(base, IDXB_C)], idx_d)
        pltpu.sync_copy(ones.at[pl.ds(0, IDXB_C)], cnt_acc.at[idx_d],
                        add=True)

    plsc.subcore_barrier()
    _dump_cnt(cnt_acc, cnt_out, c, s)


_cnt_only = pl.kernel(
    _cnt_body,
    out_type=jax.ShapeDtypeStruct((NC * CNT_PAD,), jnp.float32),
    mesh=_mesh,
    compiler_params=_sc_params,
    scratch_types=[
        pltpu.VMEM_SHARED((CNT_PAD,), jnp.float32),
        pltpu.VMEM((ZC_LEN,), jnp.float32),
        pltpu.VMEM((ONE_LEN_C,), jnp.float32),
        pltpu.VMEM((IDXB_C,), jnp.int32),
    ],
)


_spec_part = pl.BlockSpec((NC, NB, CW), lambda i: (0, i, 0))
_spec_chunk = pl.BlockSpec((NB, CW), lambda i: (i, 0))
_spec_cnt = pl.BlockSpec((NC, NB, 1), lambda i: (0, i, 0))
_spec_col = pl.BlockSpec((NB, 1), lambda i: (i, 0))
_spec_w = pl.BlockSpec((H, H), lambda i: (0, 0))
_spec_b = pl.BlockSpec((1, H), lambda i: (0, 0))
_spec_full = pl.BlockSpec((NB, H), lambda i: (i, 0))
_spec_mom = pl.BlockSpec((2, H), lambda i: (0, 0))

_chunk_out = tuple(jax.ShapeDtypeStruct((N, CW), jnp.float32)
                   for _ in range(NCH))
_spec_sage = [_spec_part] * NCH + [_spec_cnt] + [_spec_w, _spec_b, _spec_w]
_mom_out = jax.ShapeDtypeStruct((2, H), jnp.float32)
_full_out = jax.ShapeDtypeStruct((N, H), jnp.float32)


def _catx(refs):
    return jnp.concatenate([r[...] for r in refs], axis=-1)


def _catp(ps):
    return jnp.concatenate([p[0] + p[1] for p in ps], axis=-1)


def _split_store(y, outs):
    for k, o in enumerate(outs):
        o[...] = y[:, k * CW:(k + 1) * CW]


def _dotf(a, b):
    return jnp.dot(a, b, preferred_element_type=jnp.float32)


def _sage_val(ps, cnt, xd, wlT, bl, wrT):
    mean = _catp(ps) / jnp.maximum(cnt[0] + cnt[1], 1.0)
    return jnp.maximum(_dotf(mean, wlT[...]) + bl[...]
                       + _dotf(xd, wrT[...]), 0.0)


def _moments(y, i, mom_out, macc):
    s1 = jnp.sum(y, axis=0, keepdims=True)
    s2 = jnp.sum(y * y, axis=0, keepdims=True)
    @pl.when(i == 0)
    def _():
        macc[...] = jnp.zeros((2, H), jnp.float32)
    macc[0:1, :] += s1
    macc[1:2, :] += s2
    @pl.when(i == GRID - 1)
    def _():
        mom_out[...] = macc[...]


def _bn_core(y, mom, g, b):
    mu = mom[0:1, :] * (1.0 / N)
    var = mom[1:2, :] * (1.0 / N) - mu * mu
    sc = g[...] * lax.rsqrt(var + BN_EPS)
    return (y - mu) * sc + b[...]


def _sage_body(*r):
    ps = r[0:NCH]
    cnt = r[NCH]
    xd = r[NCH + 1:2 * NCH + 1]
    wlT, bl, wrT = r[2 * NCH + 1:2 * NCH + 4]
    outs = r[2 * NCH + 4:]
    _split_store(_sage_val(ps, cnt, _catx(xd), wlT, bl, wrT), outs)


def _tk_sage(parts, cnt, xd, w3):
    return pl.pallas_call(
        _sage_body,
        grid=(GRID,),
        compiler_params=_tc_params,
        in_specs=[_spec_part] * NCH + [_spec_cnt] + [_spec_chunk] * NCH
                 + [_spec_w, _spec_b, _spec_w],
        out_specs=[_spec_chunk] * NCH,
        out_shape=_chunk_out,
    )(*parts, cnt, *xd, *w3)


def _gcn_h_body(*r):
    x = r[0:NCH]
    wT = r[NCH]
    outs = r[NCH + 1:]
    _split_store(_dotf(_catx(x), wT[...]), outs)


def _tk_gcn_h(x, wT):
    return pl.pallas_call(
        _gcn_h_body,
        grid=(GRID,),
        compiler_params=_tc_params,
        in_specs=[_spec_chunk] * NCH + [_spec_w],
        out_specs=[_spec_chunk] * NCH,
        out_shape=_chunk_out,
    )(*x, wT)


def _gcn_hd_body(*r):
    x = r[0:NCH]
    wT, cnt = r[NCH], r[NCH + 1]
    outs = r[NCH + 2:NCH + 2 + NCH]
    dvo = r[2 * NCH + 2]
    dinv = lax.rsqrt(cnt[0] + cnt[1] + 1.0)
    _split_store(_dotf(_catx(x), wT[...]) * dinv, outs)
    dvo[...] = dinv


def _tk_gcn_hd(x, wT, cnt):
    return pl.pallas_call(
        _gcn_hd_body,
        grid=(GRID,),
        compiler_params=_tc_params,
        in_specs=[_spec_chunk] * NCH + [_spec_w, _spec_cnt],
        out_specs=[_spec_chunk] * NCH + [_spec_col],
        out_shape=_chunk_out + (jax.ShapeDtypeStruct((N, 1), jnp.float32),),
    )(*x, wT, cnt)


def _post_plain_body(*r):
    ps = r[0:NCH]
    b = r[NCH]
    outs = r[NCH + 1:2 * NCH + 1]
    mom = r[2 * NCH + 1]
    macc = r[2 * NCH + 2]
    i = pl.program_id(0)
    y = jnp.maximum(_catp(ps) + b[...], 0.0)
    _split_store(y, outs)
    _moments(y, i, mom, macc)


def _tk_post_plain(parts, b):
    return pl.pallas_call(
        _post_plain_body,
        grid=(GRID,),
        compiler_params=_tc_params,
        in_specs=[_spec_part] * NCH + [_spec_b],
        out_specs=[_spec_chunk] * NCH + [_spec_mom],
        out_shape=_chunk_out + (_mom_out,),
        scratch_shapes=[pltpu.VMEM((2, H), jnp.float32)],
    )(*parts, b)


def _bn_both_body(*r):
    y = r[0:NCH]
    mom, g, b = r[NCH:NCH + 3]
    full = r[NCH + 3]
    outs = r[NCH + 4:]
    out = _bn_core(_catx(y), mom, g, b)
    full[...] = out
    _split_store(out, outs)


def _tk_bn_both(y, mom, g, b):
    return pl.pallas_call(
        _bn_both_body,
        grid=(GRID,),
        compiler_params=_tc_params,
        in_specs=[_spec_chunk] * NCH + [_spec_mom, _spec_b, _spec_b],
        out_specs=[_spec_full] + [_spec_chunk] * NCH,
        out_shape=(_full_out,) + _chunk_out,
    )(*y, mom, g, b)


def _post_norm_body(*r):
    ps = r[0:NCH]
    hd = r[NCH:2 * NCH]
    dv, b = r[2 * NCH], r[2 * NCH + 1]
    outs = r[2 * NCH + 2:3 * NCH + 2]
    mom = r[3 * NCH + 2]
    macc = r[3 * NCH + 3]
    i = pl.program_id(0)
    d = dv[...]
    y = jnp.maximum(d * _catp(ps) + d * _catx(hd) + b[...], 0.0)
    _split_store(y, outs)
    _moments(y, i, mom, macc)


def _tk_post_norm(parts, hd, dv, b):
    return pl.pallas_call(
        _post_norm_body,
        grid=(GRID,),
        compiler_params=_tc_params,
        in_specs=[_spec_part] * NCH + [_spec_chunk] * NCH
                 + [_spec_col, _spec_b],
        out_specs=[_spec_chunk] * NCH + [_spec_mom],
        out_shape=_chunk_out + (_mom_out,),
        scratch_shapes=[pltpu.VMEM((2, H), jnp.float32)],
    )(*parts, *hd, dv, b)


def _bn_full_body(*r):
    y = r[0:NCH]
    mom, g, b = r[NCH:NCH + 3]
    full = r[NCH + 3]
    full[...] = _bn_core(_catx(y), mom, g, b)


def _tk_bn_full(y, mom, g, b):
    return pl.pallas_call(
        _bn_full_body,
        grid=(GRID,),
        compiler_params=_tc_params,
        in_specs=[_spec_chunk] * NCH + [_spec_mom, _spec_b, _spec_b],
        out_specs=_spec_full,
        out_shape=_full_out,
    )(*y, mom, g, b)


def _chunkn(x):
    return tuple(x[:, k * CW:(k + 1) * CW] for k in range(NCH))


def _cnt_fix(cnt_raw):
    return cnt_raw.reshape(NC, CNT_PAD)[:, :N].reshape(NC, N, 1)


def kernel(game_x, state_x, pc_x, edge_index_v_v, edge_index_history_v_s,
           edge_index_history_s_v, edge_index_in_v_s, edge_index_in_s_v,
           edge_index_s_s, edge_index_pc_pc, edge_index_pc_s,
           edge_index_s_pc, shist_sv_Wl, shist_sv_bl, shist_sv_Wr,
           sin_sv_Wl, sin_sv_bl, sin_sv_Wr, s_pc_Wl, s_pc_bl, s_pc_Wr,
           chist_vs_Wl, chist_vs_bl, chist_vs_Wr, cin_vs_Wl, cin_vs_bl,
           cin_vs_Wr, pc_s_Wl, pc_s_bl, pc_s_Wr, cfg_W, cfg_b, cfg_bn_g,
           cfg_bn_b, pc_W, pc_b, pc_bn_g, pc_bn_b, state_W, state_b,
           state_bn_g, state_bn_b):
    row = lambda v: v.reshape(1, H)
    state6 = _chunkn(state_x)
    game6 = _chunkn(game_x)
    pcx6 = _chunkn(pc_x)
    e1, e2, e3 = edge_index_history_s_v, edge_index_in_s_v, edge_index_s_pc
    e6, e7, e8 = edge_index_history_v_s, edge_index_in_v_s, edge_index_pc_s

    cnt_ss = _cnt_fix(_cnt_only(edge_index_s_s[1]))
    oa = _seg_12(*state6, e1[0], e1[1], *state6, e2[0], e2[1])
    p1, c1 = oa[0:NCH], _cnt_fix(oa[NCH])
    p2, c2 = oa[NCH + 1:2 * NCH + 1], _cnt_fix(oa[2 * NCH + 1])
    oa2 = _seg_3(*state6, e3[0], e3[1])
    p3, c3 = oa2[0:NCH], _cnt_fix(oa2[NCH])

    gx1 = _tk_sage(p1, c1, game6,
                   (shist_sv_Wl.T, row(shist_sv_bl), shist_sv_Wr.T))
    gx2 = _tk_sage(p2, c2, gx1,
                   (sin_sv_Wl.T, row(sin_sv_bl), sin_sv_Wr.T))
    px1 = _tk_sage(p3, c3, pcx6,
                   (s_pc_Wl.T, row(s_pc_bl), s_pc_Wr.T))
    hcfg = _tk_gcn_h(gx2, cfg_W.T)
    hpc = _tk_gcn_h(px1, pc_W.T)

    ob = _seg_45(*hcfg, edge_index_v_v[0], edge_index_v_v[1],
                 *hpc, edge_index_pc_pc[0], edge_index_pc_pc[1])
    p4, p5 = ob[0:NCH], ob[NCH:2 * NCH]

    oc = _tk_post_plain(p4, row(cfg_b))
    ycfg, mom_cfg = oc[0:NCH], oc[NCH]
    oc2 = _tk_post_plain(p5, row(pc_b))
    ypc, mom_pc = oc2[0:NCH], oc2[NCH]
    od = _tk_bn_both(ycfg, mom_cfg, row(cfg_bn_g), row(cfg_bn_b))
    gx_full, gx6 = od[0], od[1:NCH + 1]
    od2 = _tk_bn_both(ypc, mom_pc, row(pc_bn_g), row(pc_bn_b))
    px_full, px6 = od2[0], od2[1:NCH + 1]

    occ = _seg_67(*gx6, e6[0], e6[1], *gx6, e7[0], e7[1])
    p6, c6 = occ[0:NCH], _cnt_fix(occ[NCH])
    p7, c7 = occ[NCH + 1:2 * NCH + 1], _cnt_fix(occ[2 * NCH + 1])
    oc8 = _seg_8(*px6, e8[0], e8[1])
    p8, c8 = oc8[0:NCH], _cnt_fix(oc8[NCH])

    sx1 = _tk_sage(p6, c6, state6,
                   (chist_vs_Wl.T, row(chist_vs_bl), chist_vs_Wr.T))
    sx2 = _tk_sage(p7, c7, sx1,
                   (cin_vs_Wl.T, row(cin_vs_bl), cin_vs_Wr.T))
    sx3 = _tk_sage(p8, c8, sx2,
                   (pc_s_Wl.T, row(pc_s_bl), pc_s_Wr.T))
    oe = _tk_gcn_hd(sx3, state_W.T, cnt_ss)
    hd, dv = oe[0:NCH], oe[NCH]

    odd = _seg_9(*hd, edge_index_s_s[0], edge_index_s_s[1])
    p9 = odd[0:NCH]

    of = _tk_post_norm(p9, hd, dv, row(state_b))
    yst, mom_st = of[0:NCH], of[NCH]
    sx_full = _tk_bn_full(yst, mom_st, row(state_bn_g), row(state_bn_b))

    return (sx_full, gx_full, px_full)


# speedup vs baseline: 1.0890x; 1.0890x over previous
import functools

import jax
import jax.numpy as jnp
from jax import lax
from jax.experimental import pallas as pl
from jax.experimental.pallas import tpu as pltpu
from jax.experimental.pallas import tpu_sc as plsc

N = 50000
H = 96
E = 800000
NC, NS = 2, 16
NW = NC * NS
EPW = E // NW
BE = 200
IDXB = 5000
NBLK = EPW // IDXB
SUBB = IDXB // BE
CW = 32
NCH = H // CW
RPT = N // NS
ZR = 125
DPT = 3128
DLAST = N - (NS - 1) * DPT
CNT_PAD = 50048
CPT = CNT_PAD // NS
ZC_LEN = 800
ONE_LEN = 208

IDXB_C = 5000
ONE_LEN_C = 5008

NB = 2000
GRID = N // NB
BN_EPS = 1e-5

_mesh = plsc.VectorSubcoreMesh(core_axis_name="c", subcore_axis_name="s",
                               num_cores=NC, num_subcores=NS)
_sc_params = pltpu.CompilerParams(use_tc_tiling_on_sc=False)
_tc_params = pltpu.CompilerParams(vmem_limit_bytes=100 * 1024 * 1024)


def _zero_vmem_1d(ref, n16):
    z = jnp.zeros((16,), jnp.float32)
    @pl.loop(0, n16)
    def _(i):
        ref[pl.ds(i * 16, 16)] = z


def _zero_cnt_acc(cnt_acc, zcnt, s):
    for off in range(0, CPT, ZC_LEN):
        sz = min(ZC_LEN, CPT - off)
        pltpu.sync_copy(zcnt.at[pl.ds(0, sz)],
                        cnt_acc.at[pl.ds(s * CPT + off, sz)])


def _dump_cnt(cnt_acc, cnt_out, c, s):
    pltpu.sync_copy(cnt_acc.at[pl.ds(s * CPT, CPT)],
                    cnt_out.at[pl.ds(c * CNT_PAD + s * CPT, CPT)])


def _one_rel(with_cnt, xs, esrc, edst, aggs, cnt_out, c, s, wid,
             acc, cnt_acc, zbuf, zcnt, ones, bsrc, bdst, rbufs, sems):
    for k in range(NCH):
        for zz in range(RPT // ZR):
            pltpu.sync_copy(zbuf, acc.at[pl.ds(s * RPT + zz * ZR, ZR), :])
        if with_cnt and k == 0:
            _zero_cnt_acc(cnt_acc, zcnt, s)
        plsc.subcore_barrier()

        cnt_en = with_cnt and k == 0

        @pl.loop(0, NBLK)
        def _(blk):
            base = wid * EPW + blk * IDXB
            pltpu.sync_copy(esrc.at[pl.ds(base, IDXB)], bsrc)
            pltpu.sync_copy(edst.at[pl.ds(base, IDXB)], bdst)

            def scat(t):
                pltpu.sync_copy(rbufs[t % 2],
                                acc.at[bdst.at[pl.ds(t * BE, BE)]],
                                add=True)
                if cnt_en:
                    pltpu.sync_copy(ones.at[pl.ds(0, BE)],
                                    cnt_acc.at[bdst.at[pl.ds(t * BE, BE)]],
                                    add=True)

            descs = [None, None]
            descs[0] = pltpu.async_copy(xs[k].at[bsrc.at[pl.ds(0, BE)]],
                                        rbufs[0], sems[0])
            for t in range(1, SUBB):
                b = t % 2
                descs[b] = pltpu.async_copy(
                    xs[k].at[bsrc.at[pl.ds(t * BE, BE)]], rbufs[b], sems[b])
                descs[1 - b].wait()
                scat(t - 1)
            descs[(SUBB - 1) % 2].wait()
            scat(SUBB - 1)

        plsc.subcore_barrier()
        @pl.when(s < NS - 1)
        def _():
            pltpu.sync_copy(acc.at[pl.ds(s * DPT, DPT), :],
                            aggs[k].at[c, pl.ds(s * DPT, DPT), :])
        @pl.when(s == NS - 1)
        def _():
            pltpu.sync_copy(
                acc.at[pl.ds((NS - 1) * DPT, DLAST), :],
                aggs[k].at[c, pl.ds((NS - 1) * DPT, DLAST), :])
        if with_cnt and k == 0:
            _dump_cnt(cnt_acc, cnt_out, c, s)
        plsc.subcore_barrier()


def _cnt_rel(edst, cnt_out, c, s, wid, cnt_acc, zcnt, ones, bdst):
    _zero_cnt_acc(cnt_acc, zcnt, s)
    plsc.subcore_barrier()

    @pl.loop(0, NBLK)
    def _(blk):
        base = wid * EPW + blk * IDXB
        pltpu.sync_copy(edst.at[pl.ds(base, IDXB)], bdst)
        for t in range(SUBB):
            pltpu.sync_copy(ones.at[pl.ds(0, BE)],
                            cnt_acc.at[bdst.at[pl.ds(t * BE, BE)]],
                            add=True)

    plsc.subcore_barrier()
    _dump_cnt(cnt_acc, cnt_out, c, s)
    plsc.subcore_barrier()


def _make_seg_multi(cnt_flags, n_extra_cnt=0):
    nrel = len(cnt_flags)
    any_cnt = any(cnt_flags) or n_extra_cnt > 0
    out_types = []
    for f in cnt_flags:
        out_types += [jax.ShapeDtypeStruct((NC, N, CW), jnp.float32)] * NCH
        if f:
            out_types.append(
                jax.ShapeDtypeStruct((NC * CNT_PAD,), jnp.float32))
    out_types += [jax.ShapeDtypeStruct((NC * CNT_PAD,), jnp.float32)
                  ] * n_extra_cnt

    def body(*refs):
        pos = 0
        rels = []
        for f in cnt_flags:
            xs = refs[pos:pos + NCH]
            esrc, edst = refs[pos + NCH], refs[pos + NCH + 1]
            pos += NCH + 2
            rels.append((f, xs, esrc, edst))
        extra_edst = refs[pos:pos + n_extra_cnt]
        pos += n_extra_cnt
        outs = []
        for f in cnt_flags:
            aggs = refs[pos:pos + NCH]
            pos += NCH
            cnt_o = refs[pos] if f else None
            pos += 1 if f else 0
            outs.append((aggs, cnt_o))
        extra_cnt_out = refs[pos:pos + n_extra_cnt]
        pos += n_extra_cnt
        (acc, cnt_acc, zbuf, zcnt, ones, bsrc, bdst, rows_a, rows_b,
         sem_a, sem_b) = refs[pos:]
        c = lax.axis_index("c")
        s = lax.axis_index("s")
        wid = s * NC + c

        z = jnp.zeros((16,), jnp.float32)
        @pl.loop(0, ZR)
        def _(i):
            for j in range(CW // 16):
                zbuf[i, pl.ds(j * 16, 16)] = z
        if any_cnt:
            _zero_vmem_1d(zcnt, ZC_LEN // 16)
            o = jnp.ones((16,), jnp.float32)
            @pl.loop(0, ONE_LEN // 16)
            def _(i):
                ones[pl.ds(i * 16, 16)] = o

        for (f, xs, esrc, edst), (aggs, cnt_o) in zip(rels, outs):
            _one_rel(f, xs, esrc, edst, aggs, cnt_o, c, s, wid,
                     acc, cnt_acc, zbuf, zcnt, ones, bsrc, bdst,
                     (rows_a, rows_b), (sem_a, sem_b))
        for edst, cnt_o in zip(extra_edst, extra_cnt_out):
            _cnt_rel(edst, cnt_o, c, s, wid, cnt_acc, zcnt, ones, bdst)

    return pl.kernel(
        body,
        out_type=tuple(out_types),
        mesh=_mesh,
        compiler_params=_sc_params,
        scratch_types=[
            pltpu.VMEM_SHARED((N, CW), jnp.float32),
            pltpu.VMEM_SHARED((CNT_PAD,) if any_cnt else (8,), jnp.float32),
            pltpu.VMEM((ZR, CW), jnp.float32),
            pltpu.VMEM((ZC_LEN if any_cnt else 16,), jnp.float32),
            pltpu.VMEM((ONE_LEN if any_cnt else 16,), jnp.float32),
            pltpu.VMEM((IDXB,), jnp.int32),
            pltpu.VMEM((IDXB,), jnp.int32),
            pltpu.VMEM((BE, CW), jnp.float32),
            pltpu.VMEM((BE, CW), jnp.float32),
            pltpu.SemaphoreType.DMA,
            pltpu.SemaphoreType.DMA,
        ],
    )


_seg_1c = _make_seg_multi((True,))
_seg_1n = _make_seg_multi((False,))


def _cnt_body(edst, cnt_out, cnt_acc, zcnt, ones, idx_d):
    c = lax.axis_index("c")
    s = lax.axis_index("s")
    wid = s * NC + c
    _zero_vmem_1d(zcnt, ZC_LEN // 16)
    o = jnp.ones((16,), jnp.float32)
    @pl.loop(0, ONE_LEN_C // 16)
    def _(i):
        ones[pl.ds(i * 16, 16)] = o
    _zero_cnt_acc(cnt_acc, zcnt, s)
    plsc.subcore_barrier()

    @pl.loop(0, EPW // IDXB_C)
    def _(i):
        base = wid * EPW + i * IDXB_C
        pltpu.sync_copy(edst.at[pl.ds(base, IDXB_C)], idx_d)
        pltpu.sync_copy(ones.at[pl.ds(0, IDXB_C)], cnt_acc.at[idx_d],
                        add=True)

    plsc.subcore_barrier()
    _dump_cnt(cnt_acc, cnt_out, c, s)


_cnt_only = pl.kernel(
    _cnt_body,
    out_type=jax.ShapeDtypeStruct((NC * CNT_PAD,), jnp.float32),
    mesh=_mesh,
    compiler_params=_sc_params,
    scratch_types=[
        pltpu.VMEM_SHARED((CNT_PAD,), jnp.float32),
        pltpu.VMEM((ZC_LEN,), jnp.float32),
        pltpu.VMEM((ONE_LEN_C,), jnp.float32),
        pltpu.VMEM((IDXB_C,), jnp.int32),
    ],
)


_spec_part = pl.BlockSpec((NC, NB, CW), lambda i: (0, i, 0))
_spec_chunk = pl.BlockSpec((NB, CW), lambda i: (i, 0))
_spec_cnt = pl.BlockSpec((NC, NB, 1), lambda i: (0, i, 0))
_spec_col = pl.BlockSpec((NB, 1), lambda i: (i, 0))
_spec_w = pl.BlockSpec((H, H), lambda i: (0, 0))
_spec_b = pl.BlockSpec((1, H), lambda i: (0, 0))
_spec_full = pl.BlockSpec((NB, H), lambda i: (i, 0))
_spec_mom = pl.BlockSpec((2, H), lambda i: (0, 0))

_chunk_out = tuple(jax.ShapeDtypeStruct((N, CW), jnp.float32)
                   for _ in range(NCH))
_spec_sage = [_spec_part] * NCH + [_spec_cnt] + [_spec_w, _spec_b, _spec_w]
_mom_out = jax.ShapeDtypeStruct((2, H), jnp.float32)
_full_out = jax.ShapeDtypeStruct((N, H), jnp.float32)


def _catx(refs):
    return jnp.concatenate([r[...] for r in refs], axis=-1)


def _catp(ps):
    return jnp.concatenate([p[0] + p[1] for p in ps], axis=-1)


def _split_store(y, outs):
    for k, o in enumerate(outs):
        o[...] = y[:, k * CW:(k + 1) * CW]


def _dotf(a, b):
    return jnp.dot(a, b, preferred_element_type=jnp.float32)


def _sage_val(ps, cnt, xd, wlT, bl, wrT):
    mean = _catp(ps) / jnp.maximum(cnt[0] + cnt[1], 1.0)
    return jnp.maximum(_dotf(mean, wlT[...]) + bl[...]
                       + _dotf(xd, wrT[...]), 0.0)


def _moments(y, i, mom_out, macc):
    s1 = jnp.sum(y, axis=0, keepdims=True)
    s2 = jnp.sum(y * y, axis=0, keepdims=True)
    @pl.when(i == 0)
    def _():
        macc[...] = jnp.zeros((2, H), jnp.float32)
    macc[0:1, :] += s1
    macc[1:2, :] += s2
    @pl.when(i == GRID - 1)
    def _():
        mom_out[...] = macc[...]


def _bn_core(y, mom, g, b):
    mu = mom[0:1, :] * (1.0 / N)
    var = mom[1:2, :] * (1.0 / N) - mu * mu
    sc = g[...] * lax.rsqrt(var + BN_EPS)
    return (y - mu) * sc + b[...]


def _sage_body(*r):
    ps = r[0:NCH]
    cnt = r[NCH]
    xd = r[NCH + 1:2 * NCH + 1]
    wlT, bl, wrT = r[2 * NCH + 1:2 * NCH + 4]
    outs = r[2 * NCH + 4:]
    _split_store(_sage_val(ps, cnt, _catx(xd), wlT, bl, wrT), outs)


def _tk_sage(parts, cnt, xd, w3):
    return pl.pallas_call(
        _sage_body,
        grid=(GRID,),
        compiler_params=_tc_params,
        in_specs=[_spec_part] * NCH + [_spec_cnt] + [_spec_chunk] * NCH
                 + [_spec_w, _spec_b, _spec_w],
        out_specs=[_spec_chunk] * NCH,
        out_shape=_chunk_out,
    )(*parts, cnt, *xd, *w3)


def _gcn_h_body(*r):
    x = r[0:NCH]
    wT = r[NCH]
    outs = r[NCH + 1:]
    _split_store(_dotf(_catx(x), wT[...]), outs)


def _tk_gcn_h(x, wT):
    return pl.pallas_call(
        _gcn_h_body,
        grid=(GRID,),
        compiler_params=_tc_params,
        in_specs=[_spec_chunk] * NCH + [_spec_w],
        out_specs=[_spec_chunk] * NCH,
        out_shape=_chunk_out,
    )(*x, wT)


def _gcn_hd_body(*r):
    x = r[0:NCH]
    wT, cnt = r[NCH], r[NCH + 1]
    outs = r[NCH + 2:NCH + 2 + NCH]
    dvo = r[2 * NCH + 2]
    dinv = lax.rsqrt(cnt[0] + cnt[1] + 1.0)
    _split_store(_dotf(_catx(x), wT[...]) * dinv, outs)
    dvo[...] = dinv


def _tk_gcn_hd(x, wT, cnt):
    return pl.pallas_call(
        _gcn_hd_body,
        grid=(GRID,),
        compiler_params=_tc_params,
        in_specs=[_spec_chunk] * NCH + [_spec_w, _spec_cnt],
        out_specs=[_spec_chunk] * NCH + [_spec_col],
        out_shape=_chunk_out + (jax.ShapeDtypeStruct((N, 1), jnp.float32),),
    )(*x, wT, cnt)


def _post_plain_body(*r):
    ps = r[0:NCH]
    b = r[NCH]
    outs = r[NCH + 1:2 * NCH + 1]
    mom = r[2 * NCH + 1]
    macc = r[2 * NCH + 2]
    i = pl.program_id(0)
    y = jnp.maximum(_catp(ps) + b[...], 0.0)
    _split_store(y, outs)
    _moments(y, i, mom, macc)


def _tk_post_plain(parts, b):
    return pl.pallas_call(
        _post_plain_body,
        grid=(GRID,),
        compiler_params=_tc_params,
        in_specs=[_spec_part] * NCH + [_spec_b],
        out_specs=[_spec_chunk] * NCH + [_spec_mom],
        out_shape=_chunk_out + (_mom_out,),
        scratch_shapes=[pltpu.VMEM((2, H), jnp.float32)],
    )(*parts, b)


def _bn_both_body(*r):
    y = r[0:NCH]
    mom, g, b = r[NCH:NCH + 3]
    full = r[NCH + 3]
    outs = r[NCH + 4:]
    out = _bn_core(_catx(y), mom, g, b)
    full[...] = out
    _split_store(out, outs)


def _tk_bn_both(y, mom, g, b):
    return pl.pallas_call(
        _bn_both_body,
        grid=(GRID,),
        compiler_params=_tc_params,
        in_specs=[_spec_chunk] * NCH + [_spec_mom, _spec_b, _spec_b],
        out_specs=[_spec_full] + [_spec_chunk] * NCH,
        out_shape=(_full_out,) + _chunk_out,
    )(*y, mom, g, b)


def _post_norm_body(*r):
    ps = r[0:NCH]
    hd = r[NCH:2 * NCH]
    dv, b = r[2 * NCH], r[2 * NCH + 1]
    outs = r[2 * NCH + 2:3 * NCH + 2]
    mom = r[3 * NCH + 2]
    macc = r[3 * NCH + 3]
    i = pl.program_id(0)
    d = dv[...]
    y = jnp.maximum(d * _catp(ps) + d * _catx(hd) + b[...], 0.0)
    _split_store(y, outs)
    _moments(y, i, mom, macc)


def _tk_post_norm(parts, hd, dv, b):
    return pl.pallas_call(
        _post_norm_body,
        grid=(GRID,),
        compiler_params=_tc_params,
        in_specs=[_spec_part] * NCH + [_spec_chunk] * NCH
                 + [_spec_col, _spec_b],
        out_specs=[_spec_chunk] * NCH + [_spec_mom],
        out_shape=_chunk_out + (_mom_out,),
        scratch_shapes=[pltpu.VMEM((2, H), jnp.float32)],
    )(*parts, *hd, dv, b)


def _bn_full_body(*r):
    y = r[0:NCH]
    mom, g, b = r[NCH:NCH + 3]
    full = r[NCH + 3]
    full[...] = _bn_core(_catx(y), mom, g, b)


def _tk_bn_full(y, mom, g, b):
    return pl.pallas_call(
        _bn_full_body,
        grid=(GRID,),
        compiler_params=_tc_params,
        in_specs=[_spec_chunk] * NCH + [_spec_mom, _spec_b, _spec_b],
        out_specs=_spec_full,
        out_shape=_full_out,
    )(*y, mom, g, b)


def _chunkn(x):
    return tuple(x[:, k * CW:(k + 1) * CW] for k in range(NCH))


def _cnt_fix(cnt_raw):
    return cnt_raw.reshape(NC, CNT_PAD)[:, :N].reshape(NC, N, 1)


def kernel(game_x, state_x, pc_x, edge_index_v_v, edge_index_history_v_s,
           edge_index_history_s_v, edge_index_in_v_s, edge_index_in_s_v,
           edge_index_s_s, edge_index_pc_pc, edge_index_pc_s,
           edge_index_s_pc, shist_sv_Wl, shist_sv_bl, shist_sv_Wr,
           sin_sv_Wl, sin_sv_bl, sin_sv_Wr, s_pc_Wl, s_pc_bl, s_pc_Wr,
           chist_vs_Wl, chist_vs_bl, chist_vs_Wr, cin_vs_Wl, cin_vs_bl,
           cin_vs_Wr, pc_s_Wl, pc_s_bl, pc_s_Wr, cfg_W, cfg_b, cfg_bn_g,
           cfg_bn_b, pc_W, pc_b, pc_bn_g, pc_bn_b, state_W, state_b,
           state_bn_g, state_bn_b):
    row = lambda v: v.reshape(1, H)
    state6 = _chunkn(state_x)
    game6 = _chunkn(game_x)
    pcx6 = _chunkn(pc_x)
    e1, e2, e3 = edge_index_history_s_v, edge_index_in_s_v, edge_index_s_pc
    e6, e7, e8 = edge_index_history_v_s, edge_index_in_v_s, edge_index_pc_s

    cnt_ss = _cnt_fix(_cnt_only(edge_index_s_s[1]))
    oa = _seg_1c(*state6, e1[0], e1[1])
    p1, c1 = oa[0:NCH], _cnt_fix(oa[NCH])
    oa2 = _seg_1c(*state6, e2[0], e2[1])
    p2, c2 = oa2[0:NCH], _cnt_fix(oa2[NCH])
    oa3 = _seg_1c(*state6, e3[0], e3[1])
    p3, c3 = oa3[0:NCH], _cnt_fix(oa3[NCH])

    gx1 = _tk_sage(p1, c1, game6,
                   (shist_sv_Wl.T, row(shist_sv_bl), shist_sv_Wr.T))
    gx2 = _tk_sage(p2, c2, gx1,
                   (sin_sv_Wl.T, row(sin_sv_bl), sin_sv_Wr.T))
    px1 = _tk_sage(p3, c3, pcx6,
                   (s_pc_Wl.T, row(s_pc_bl), s_pc_Wr.T))
    hcfg = _tk_gcn_h(gx2, cfg_W.T)
    hpc = _tk_gcn_h(px1, pc_W.T)

    ob = _seg_1n(*hcfg, edge_index_v_v[0], edge_index_v_v[1])
    p4 = ob[0:NCH]
    ob2 = _seg_1n(*hpc, edge_index_pc_pc[0], edge_index_pc_pc[1])
    p5 = ob2[0:NCH]

    oc = _tk_post_plain(p4, row(cfg_b))
    ycfg, mom_cfg = oc[0:NCH], oc[NCH]
    oc2 = _tk_post_plain(p5, row(pc_b))
    ypc, mom_pc = oc2[0:NCH], oc2[NCH]
    od = _tk_bn_both(ycfg, mom_cfg, row(cfg_bn_g), row(cfg_bn_b))
    gx_full, gx6 = od[0], od[1:NCH + 1]
    od2 = _tk_bn_both(ypc, mom_pc, row(pc_bn_g), row(pc_bn_b))
    px_full, px6 = od2[0], od2[1:NCH + 1]

    occ = _seg_1c(*gx6, e6[0], e6[1])
    p6, c6 = occ[0:NCH], _cnt_fix(occ[NCH])
    oc7 = _seg_1c(*gx6, e7[0], e7[1])
    p7, c7 = oc7[0:NCH], _cnt_fix(oc7[NCH])
    oc8 = _seg_1c(*px6, e8[0], e8[1])
    p8, c8 = oc8[0:NCH], _cnt_fix(oc8[NCH])

    sx1 = _tk_sage(p6, c6, state6,
                   (chist_vs_Wl.T, row(chist_vs_bl), chist_vs_Wr.T))
    sx2 = _tk_sage(p7, c7, sx1,
                   (cin_vs_Wl.T, row(cin_vs_bl), cin_vs_Wr.T))
    sx3 = _tk_sage(p8, c8, sx2,
                   (pc_s_Wl.T, row(pc_s_bl), pc_s_Wr.T))
    oe = _tk_gcn_hd(sx3, state_W.T, cnt_ss)
    hd, dv = oe[0:NCH], oe[NCH]

    odd = _seg_1n(*hd, edge_index_s_s[0], edge_index_s_s[1])
    p9 = odd[0:NCH]

    of = _tk_post_norm(p9, hd, dv, row(state_b))
    yst, mom_st = of[0:NCH], of[NCH]
    sx_full = _tk_bn_full(yst, mom_st, row(state_bn_g), row(state_bn_b))

    return (sx_full, gx_full, px_full)


# speedup vs baseline: 1.0899x; 1.0009x over previous
import jax
import jax.numpy as jnp
from jax import lax
from jax.experimental import pallas as pl
from jax.experimental.pallas import tpu as pltpu
from jax.experimental.pallas import tpu_sc as plsc

N = 50000
H = 96
E = 800000
NC, NS = 2, 16
NW = NC * NS
EPW = E // NW
BE = 200
IDXB = 5000
NBLK = EPW // IDXB
SUBB = IDXB // BE
CW = 32
NCH = H // CW
RPT = N // NS
ZR = 125
DPT = 3128
DLAST = N - (NS - 1) * DPT
CNT_PAD = 50048
CPT = CNT_PAD // NS
ZC_LEN = 800
ONE_LEN = 208

IDXB_C = 5000
ONE_LEN_C = 5008

NB = 2000
GRID = N // NB
BN_EPS = 1e-5

_mesh = plsc.VectorSubcoreMesh(core_axis_name="c", subcore_axis_name="s",
                               num_cores=NC, num_subcores=NS)
_sc_params = pltpu.CompilerParams(use_tc_tiling_on_sc=False)
_tc_params = pltpu.CompilerParams(vmem_limit_bytes=100 * 1024 * 1024)


def _zero_vmem_1d(ref, n16):
    z = jnp.zeros((16,), jnp.float32)
    @pl.loop(0, n16)
    def _(i):
        ref[pl.ds(i * 16, 16)] = z


def _zero_cnt_acc(cnt_acc, zcnt, s):
    for off in range(0, CPT, ZC_LEN):
        sz = min(ZC_LEN, CPT - off)
        pltpu.sync_copy(zcnt.at[pl.ds(0, sz)],
                        cnt_acc.at[pl.ds(s * CPT + off, sz)])


def _dump_cnt(cnt_acc, cnt_out, c, s):
    pltpu.sync_copy(cnt_acc.at[pl.ds(s * CPT, CPT)],
                    cnt_out.at[pl.ds(c * CNT_PAD + s * CPT, CPT)])


def _one_rel(with_cnt, xs, esrc, edst, aggs, cnt_out, c, s, wid,
             acc, cnt_acc, zbuf, zcnt, ones, bsrc, bdst, rbufs, sems):
    for k in range(NCH):
        for zz in range(RPT // ZR):
            pltpu.sync_copy(zbuf, acc.at[pl.ds(s * RPT + zz * ZR, ZR), :])
        if with_cnt and k == 0:
            _zero_cnt_acc(cnt_acc, zcnt, s)
        plsc.subcore_barrier()

        cnt_en = with_cnt and k == 0

        @pl.loop(0, NBLK)
        def _(blk):
            base = wid * EPW + blk * IDXB
            pltpu.sync_copy(esrc.at[pl.ds(base, IDXB)], bsrc)
            pltpu.sync_copy(edst.at[pl.ds(base, IDXB)], bdst)

            def scat(t):
                pltpu.sync_copy(rbufs[t % 2],
                                acc.at[bdst.at[pl.ds(t * BE, BE)]],
                                add=True)
                if cnt_en:
                    pltpu.sync_copy(ones.at[pl.ds(0, BE)],
                                    cnt_acc.at[bdst.at[pl.ds(t * BE, BE)]],
                                    add=True)

            descs = [None, None]
            descs[0] = pltpu.async_copy(xs[k].at[bsrc.at[pl.ds(0, BE)]],
                                        rbufs[0], sems[0])
            for t in range(1, SUBB):
                b = t % 2
                descs[b] = pltpu.async_copy(
                    xs[k].at[bsrc.at[pl.ds(t * BE, BE)]], rbufs[b], sems[b])
                descs[1 - b].wait()
                scat(t - 1)
            descs[(SUBB - 1) % 2].wait()
            scat(SUBB - 1)

        plsc.subcore_barrier()
        @pl.when(s < NS - 1)
        def _():
            pltpu.sync_copy(acc.at[pl.ds(s * DPT, DPT), :],
                            aggs[k].at[c, pl.ds(s * DPT, DPT), :])
        @pl.when(s == NS - 1)
        def _():
            pltpu.sync_copy(
                acc.at[pl.ds((NS - 1) * DPT, DLAST), :],
                aggs[k].at[c, pl.ds((NS - 1) * DPT, DLAST), :])
        if with_cnt and k == 0:
            _dump_cnt(cnt_acc, cnt_out, c, s)
        plsc.subcore_barrier()


def _cnt_rel(edst, cnt_out, c, s, wid, cnt_acc, zcnt, ones, bdst):
    _zero_cnt_acc(cnt_acc, zcnt, s)
    plsc.subcore_barrier()

    @pl.loop(0, NBLK)
    def _(blk):
        base = wid * EPW + blk * IDXB
        pltpu.sync_copy(edst.at[pl.ds(base, IDXB)], bdst)
        for t in range(SUBB):
            pltpu.sync_copy(ones.at[pl.ds(0, BE)],
                            cnt_acc.at[bdst.at[pl.ds(t * BE, BE)]],
                            add=True)

    plsc.subcore_barrier()
    _dump_cnt(cnt_acc, cnt_out, c, s)
    plsc.subcore_barrier()


def _make_seg_multi(cnt_flags, n_extra_cnt=0):
    nrel = len(cnt_flags)
    any_cnt = any(cnt_flags) or n_extra_cnt > 0
    out_types = []
    for f in cnt_flags:
        out_types += [jax.ShapeDtypeStruct((NC, N, CW), jnp.float32)] * NCH
        if f:
            out_types.append(
                jax.ShapeDtypeStruct((NC * CNT_PAD,), jnp.float32))
    out_types += [jax.ShapeDtypeStruct((NC * CNT_PAD,), jnp.float32)
                  ] * n_extra_cnt

    def body(*refs):
        pos = 0
        rels = []
        for f in cnt_flags:
            xs = refs[pos:pos + NCH]
            esrc, edst = refs[pos + NCH], refs[pos + NCH + 1]
            pos += NCH + 2
            rels.append((f, xs, esrc, edst))
        extra_edst = refs[pos:pos + n_extra_cnt]
        pos += n_extra_cnt
        outs = []
        for f in cnt_flags:
            aggs = refs[pos:pos + NCH]
            pos += NCH
            cnt_o = refs[pos] if f else None
            pos += 1 if f else 0
            outs.append((aggs, cnt_o))
        extra_cnt_out = refs[pos:pos + n_extra_cnt]
        pos += n_extra_cnt
        (acc, cnt_acc, zbuf, zcnt, ones, bsrc, bdst, rows_a, rows_b,
         sem_a, sem_b) = refs[pos:]
        c = lax.axis_index("c")
        s = lax.axis_index("s")
        wid = s * NC + c

        z = jnp.zeros((16,), jnp.float32)
        @pl.loop(0, ZR)
        def _(i):
            for j in range(CW // 16):
                zbuf[i, pl.ds(j * 16, 16)] = z
        if any_cnt:
            _zero_vmem_1d(zcnt, ZC_LEN // 16)
            o = jnp.ones((16,), jnp.float32)
            @pl.loop(0, ONE_LEN // 16)
            def _(i):
                ones[pl.ds(i * 16, 16)] = o

        for (f, xs, esrc, edst), (aggs, cnt_o) in zip(rels, outs):
            _one_rel(f, xs, esrc, edst, aggs, cnt_o, c, s, wid,
                     acc, cnt_acc, zbuf, zcnt, ones, bsrc, bdst,
                     (rows_a, rows_b), (sem_a, sem_b))
        for edst, cnt_o in zip(extra_edst, extra_cnt_out):
            _cnt_rel(edst, cnt_o, c, s, wid, cnt_acc, zcnt, ones, bdst)

    return pl.kernel(
        body,
        out_type=tuple(out_types),
        mesh=_mesh,
        compiler_params=_sc_params,
        scratch_types=[
            pltpu.VMEM_SHARED((N, CW), jnp.float32),
            pltpu.VMEM_SHARED((CNT_PAD,) if any_cnt else (8,), jnp.float32),
            pltpu.VMEM((ZR, CW), jnp.float32),
            pltpu.VMEM((ZC_LEN if any_cnt else 16,), jnp.float32),
            pltpu.VMEM((ONE_LEN if any_cnt else 16,), jnp.float32),
            pltpu.VMEM((IDXB,), jnp.int32),
            pltpu.VMEM((IDXB,), jnp.int32),
            pltpu.VMEM((BE, CW), jnp.float32),
            pltpu.VMEM((BE, CW), jnp.float32),
            pltpu.SemaphoreType.DMA,
            pltpu.SemaphoreType.DMA,
        ],
    )


_seg_1c = _make_seg_multi((True,))
_seg_1n = _make_seg_multi((False,))


def _cnt_body(edst, cnt_out, cnt_acc, zcnt, ones, idx_d):
    c = lax.axis_index("c")
    s = lax.axis_index("s")
    wid = s * NC + c
    _zero_vmem_1d(zcnt, ZC_LEN // 16)
    o = jnp.ones((16,), jnp.float32)
    @pl.loop(0, ONE_LEN_C // 16)
    def _(i):
        ones[pl.ds(i * 16, 16)] = o
    _zero_cnt_acc(cnt_acc, zcnt, s)
    plsc.subcore_barrier()

    @pl.loop(0, EPW // IDXB_C)
    def _(i):
        base = wid * EPW + i * IDXB_C
        pltpu.sync_copy(edst.at[pl.ds(base, IDXB_C)], idx_d)
        pltpu.sync_copy(ones.at[pl.ds(0, IDXB_C)], cnt_acc.at[idx_d],
                        add=True)

    plsc.subcore_barrier()
    _dump_cnt(cnt_acc, cnt_out, c, s)


_cnt_only = pl.kernel(
    _cnt_body,
    out_type=jax.ShapeDtypeStruct((NC * CNT_PAD,), jnp.float32),
    mesh=_mesh,
    compiler_params=_sc_params,
    scratch_types=[
        pltpu.VMEM_SHARED((CNT_PAD,), jnp.float32),
        pltpu.VMEM((ZC_LEN,), jnp.float32),
        pltpu.VMEM((ONE_LEN_C,), jnp.float32),
        pltpu.VMEM((IDXB_C,), jnp.int32),
    ],
)


_spec_part = pl.BlockSpec((NC, NB, CW), lambda i: (0, i, 0))
_spec_chunk = pl.BlockSpec((NB, CW), lambda i: (i, 0))
_spec_cnt = pl.BlockSpec((NC, NB, 1), lambda i: (0, i, 0))
_spec_col = pl.BlockSpec((NB, 1), lambda i: (i, 0))
_spec_w = pl.BlockSpec((H, H), lambda i: (0, 0))
_spec_b = pl.BlockSpec((1, H), lambda i: (0, 0))
_spec_full = pl.BlockSpec((NB, H), lambda i: (i, 0))
_spec_mom = pl.BlockSpec((2, H), lambda i: (0, 0))

_chunk_out = tuple(jax.ShapeDtypeStruct((N, CW), jnp.float32)
                   for _ in range(NCH))
_spec_sage = [_spec_part] * NCH + [_spec_cnt] + [_spec_w, _spec_b, _spec_w]
_mom_out = jax.ShapeDtypeStruct((2, H), jnp.float32)
_full_out = jax.ShapeDtypeStruct((N, H), jnp.float32)


def _catx(refs):
    return jnp.concatenate([r[...] for r in refs], axis=-1)


def _catp(ps):
    return jnp.concatenate([p[0] + p[1] for p in ps], axis=-1)


def _split_store(y, outs):
    for k, o in enumerate(outs):
        o[...] = y[:, k * CW:(k + 1) * CW]


def _dotf(a, b):
    return jnp.dot(a, b, preferred_element_type=jnp.float32)


def _sage_val(ps, cnt, xd, wlT, bl, wrT):
    mean = _catp(ps) / jnp.maximum(cnt[0] + cnt[1], 1.0)
    return jnp.maximum(_dotf(mean, wlT[...]) + bl[...]
                       + _dotf(xd, wrT[...]), 0.0)


def _moments(y, i, mom_out, macc):
    s1 = jnp.sum(y, axis=0, keepdims=True)
    s2 = jnp.sum(y * y, axis=0, keepdims=True)
    @pl.when(i == 0)
    def _():
        macc[...] = jnp.zeros((2, H), jnp.float32)
    macc[0:1, :] += s1
    macc[1:2, :] += s2
    @pl.when(i == GRID - 1)
    def _():
        mom_out[...] = macc[...]


def _bn_core(y, mom, g, b):
    mu = mom[0:1, :] * (1.0 / N)
    var = mom[1:2, :] * (1.0 / N) - mu * mu
    sc = g[...] * lax.rsqrt(var + BN_EPS)
    return (y - mu) * sc + b[...]


def _sage_body(*r):
    ps = r[0:NCH]
    cnt = r[NCH]
    xd = r[NCH + 1:2 * NCH + 1]
    wlT, bl, wrT = r[2 * NCH + 1:2 * NCH + 4]
    outs = r[2 * NCH + 4:]
    _split_store(_sage_val(ps, cnt, _catx(xd), wlT, bl, wrT), outs)


def _tk_sage(parts, cnt, xd, w3):
    return pl.pallas_call(
        _sage_body,
        grid=(GRID,),
        compiler_params=_tc_params,
        in_specs=[_spec_part] * NCH + [_spec_cnt] + [_spec_chunk] * NCH
                 + [_spec_w, _spec_b, _spec_w],
        out_specs=[_spec_chunk] * NCH,
        out_shape=_chunk_out,
    )(*parts, cnt, *xd, *w3)


def _gcn_h_body(*r):
    x = r[0:NCH]
    wT = r[NCH]
    outs = r[NCH + 1:]
    _split_store(_dotf(_catx(x), wT[...]), outs)


def _tk_gcn_h(x, wT):
    return pl.pallas_call(
        _gcn_h_body,
        grid=(GRID,),
        compiler_params=_tc_params,
        in_specs=[_spec_chunk] * NCH + [_spec_w],
        out_specs=[_spec_chunk] * NCH,
        out_shape=_chunk_out,
    )(*x, wT)


def _gcn_hd_body(*r):
    x = r[0:NCH]
    wT, cnt = r[NCH], r[NCH + 1]
    outs = r[NCH + 2:NCH + 2 + NCH]
    dvo = r[2 * NCH + 2]
    dinv = lax.rsqrt(cnt[0] + cnt[1] + 1.0)
    _split_store(_dotf(_catx(x), wT[...]) * dinv, outs)
    dvo[...] = dinv


def _tk_gcn_hd(x, wT, cnt):
    return pl.pallas_call(
        _gcn_hd_body,
        grid=(GRID,),
        compiler_params=_tc_params,
        in_specs=[_spec_chunk] * NCH + [_spec_w, _spec_cnt],
        out_specs=[_spec_chunk] * NCH + [_spec_col],
        out_shape=_chunk_out + (jax.ShapeDtypeStruct((N, 1), jnp.float32),),
    )(*x, wT, cnt)


def _post_plain_body(*r):
    ps = r[0:NCH]
    b = r[NCH]
    outs = r[NCH + 1:2 * NCH + 1]
    mom = r[2 * NCH + 1]
    macc = r[2 * NCH + 2]
    i = pl.program_id(0)
    y = jnp.maximum(_catp(ps) + b[...], 0.0)
    _split_store(y, outs)
    _moments(y, i, mom, macc)


def _tk_post_plain(parts, b):
    return pl.pallas_call(
        _post_plain_body,
        grid=(GRID,),
        compiler_params=_tc_params,
        in_specs=[_spec_part] * NCH + [_spec_b],
        out_specs=[_spec_chunk] * NCH + [_spec_mom],
        out_shape=_chunk_out + (_mom_out,),
        scratch_shapes=[pltpu.VMEM((2, H), jnp.float32)],
    )(*parts, b)


def _bn_both_body(*r):
    y = r[0:NCH]
    mom, g, b = r[NCH:NCH + 3]
    full = r[NCH + 3]
    outs = r[NCH + 4:]
    out = _bn_core(_catx(y), mom, g, b)
    full[...] = out
    _split_store(out, outs)


def _tk_bn_both(y, mom, g, b):
    return pl.pallas_call(
        _bn_both_body,
        grid=(GRID,),
        compiler_params=_tc_params,
        in_specs=[_spec_chunk] * NCH + [_spec_mom, _spec_b, _spec_b],
        out_specs=[_spec_full] + [_spec_chunk] * NCH,
        out_shape=(_full_out,) + _chunk_out,
    )(*y, mom, g, b)


def _post_norm_body(*r):
    ps = r[0:NCH]
    hd = r[NCH:2 * NCH]
    dv, b = r[2 * NCH], r[2 * NCH + 1]
    outs = r[2 * NCH + 2:3 * NCH + 2]
    mom = r[3 * NCH + 2]
    macc = r[3 * NCH + 3]
    i = pl.program_id(0)
    d = dv[...]
    y = jnp.maximum(d * _catp(ps) + d * _catx(hd) + b[...], 0.0)
    _split_store(y, outs)
    _moments(y, i, mom, macc)


def _tk_post_norm(parts, hd, dv, b):
    return pl.pallas_call(
        _post_norm_body,
        grid=(GRID,),
        compiler_params=_tc_params,
        in_specs=[_spec_part] * NCH + [_spec_chunk] * NCH
                 + [_spec_col, _spec_b],
        out_specs=[_spec_chunk] * NCH + [_spec_mom],
        out_shape=_chunk_out + (_mom_out,),
        scratch_shapes=[pltpu.VMEM((2, H), jnp.float32)],
    )(*parts, *hd, dv, b)


def _bn_full_body(*r):
    y = r[0:NCH]
    mom, g, b = r[NCH:NCH + 3]
    full = r[NCH + 3]
    full[...] = _bn_core(_catx(y), mom, g, b)


def _tk_bn_full(y, mom, g, b):
    return pl.pallas_call(
        _bn_full_body,
        grid=(GRID,),
        compiler_params=_tc_params,
        in_specs=[_spec_chunk] * NCH + [_spec_mom, _spec_b, _spec_b],
        out_specs=_spec_full,
        out_shape=_full_out,
    )(*y, mom, g, b)


def _chunkn(x):
    return tuple(x[:, k * CW:(k + 1) * CW] for k in range(NCH))


def _cnt_fix(cnt_raw):
    return cnt_raw.reshape(NC, CNT_PAD)[:, :N].reshape(NC, N, 1)


def kernel(game_x, state_x, pc_x, edge_index_v_v, edge_index_history_v_s,
           edge_index_history_s_v, edge_index_in_v_s, edge_index_in_s_v,
           edge_index_s_s, edge_index_pc_pc, edge_index_pc_s,
           edge_index_s_pc, shist_sv_Wl, shist_sv_bl, shist_sv_Wr,
           sin_sv_Wl, sin_sv_bl, sin_sv_Wr, s_pc_Wl, s_pc_bl, s_pc_Wr,
           chist_vs_Wl, chist_vs_bl, chist_vs_Wr, cin_vs_Wl, cin_vs_bl,
           cin_vs_Wr, pc_s_Wl, pc_s_bl, pc_s_Wr, cfg_W, cfg_b, cfg_bn_g,
           cfg_bn_b, pc_W, pc_b, pc_bn_g, pc_bn_b, state_W, state_b,
           state_bn_g, state_bn_b):
    row = lambda v: v.reshape(1, H)
    state6 = _chunkn(state_x)
    game6 = _chunkn(game_x)
    pcx6 = _chunkn(pc_x)
    e1, e2, e3 = edge_index_history_s_v, edge_index_in_s_v, edge_index_s_pc
    e6, e7, e8 = edge_index_history_v_s, edge_index_in_v_s, edge_index_pc_s

    cnt_ss = _cnt_fix(_cnt_only(edge_index_s_s[1]))
    oa = _seg_1c(*state6, e1[0], e1[1])
    p1, c1 = oa[0:NCH], _cnt_fix(oa[NCH])
    oa2 = _seg_1c(*state6, e2[0], e2[1])
    p2, c2 = oa2[0:NCH], _cnt_fix(oa2[NCH])
    oa3 = _seg_1c(*state6, e3[0], e3[1])
    p3, c3 = oa3[0:NCH], _cnt_fix(oa3[NCH])

    gx1 = _tk_sage(p1, c1, game6,
                   (shist_sv_Wl.T, row(shist_sv_bl), shist_sv_Wr.T))
    gx2 = _tk_sage(p2, c2, gx1,
                   (sin_sv_Wl.T, row(sin_sv_bl), sin_sv_Wr.T))
    px1 = _tk_sage(p3, c3, pcx6,
                   (s_pc_Wl.T, row(s_pc_bl), s_pc_Wr.T))
    hcfg = _tk_gcn_h(gx2, cfg_W.T)
    hpc = _tk_gcn_h(px1, pc_W.T)

    ob = _seg_1n(*hcfg, edge_index_v_v[0], edge_index_v_v[1])
    p4 = ob[0:NCH]
    ob2 = _seg_1n(*hpc, edge_index_pc_pc[0], edge_index_pc_pc[1])
    p5 = ob2[0:NCH]

    oc = _tk_post_plain(p4, row(cfg_b))
    ycfg, mom_cfg = oc[0:NCH], oc[NCH]
    oc2 = _tk_post_plain(p5, row(pc_b))
    ypc, mom_pc = oc2[0:NCH], oc2[NCH]
    od = _tk_bn_both(ycfg, mom_cfg, row(cfg_bn_g), row(cfg_bn_b))
    gx_full, gx6 = od[0], od[1:NCH + 1]
    od2 = _tk_bn_both(ypc, mom_pc, row(pc_bn_g), row(pc_bn_b))
    px_full, px6 = od2[0], od2[1:NCH + 1]

    occ = _seg_1c(*gx6, e6[0], e6[1])
    p6, c6 = occ[0:NCH], _cnt_fix(occ[NCH])
    oc7 = _seg_1c(*gx6, e7[0], e7[1])
    p7, c7 = oc7[0:NCH], _cnt_fix(oc7[NCH])
    oc8 = _seg_1c(*px6, e8[0], e8[1])
    p8, c8 = oc8[0:NCH], _cnt_fix(oc8[NCH])

    sx1 = _tk_sage(p6, c6, state6,
                   (chist_vs_Wl.T, row(chist_vs_bl), chist_vs_Wr.T))
    sx2 = _tk_sage(p7, c7, sx1,
                   (cin_vs_Wl.T, row(cin_vs_bl), cin_vs_Wr.T))
    sx3 = _tk_sage(p8, c8, sx2,
                   (pc_s_Wl.T, row(pc_s_bl), pc_s_Wr.T))
    oe = _tk_gcn_hd(sx3, state_W.T, cnt_ss)
    hd, dv = oe[0:NCH], oe[NCH]

    odd = _seg_1n(*hd, edge_index_s_s[0], edge_index_s_s[1])
    p9 = odd[0:NCH]

    of = _tk_post_norm(p9, hd, dv, row(state_b))
    yst, mom_st = of[0:NCH], of[NCH]
    sx_full = _tk_bn_full(yst, mom_st, row(state_bn_g), row(state_bn_b))

    return (sx_full, gx_full, px_full)
